# baseline (speedup 1.0000x reference)
import functools
import math

import jax
import jax.numpy as jnp
from jax import lax
from jax.experimental import pallas as pl
from jax.experimental.pallas import tpu as pltpu
from jax.experimental.pallas import tpu_sc as plsc

N = 10000
E = 160000
C = 256
HALF = 128
ALPHA = 0.5
BETA = math.log(2.0)

NC = 2
NS = 16
LANES = 16
CHUNK = 128
N_CHUNKS = E // CHUNK
ROWS_PER_TILE = 624
TAIL_ROWS = N - NS * ROWS_PER_TILE
ZROWS = 208

_MESH = plsc.VectorSubcoreMesh(core_axis_name="c", subcore_axis_name="s")


def _zero_f32_ref(ref, nwords):
  zeros = jnp.zeros((LANES,), jnp.float32)

  @pl.loop(0, nwords // LANES)
  def _(i):
    ref[pl.ds(i * LANES, LANES)] = zeros


_ZCH = 640
_DB = 8


def _deg_body(edge_hbm, ew_hbm, deg2_hbm, cb, wb, zbuf_v, deg_sh, isem, ssem):
  cid = lax.axis_index("c")
  sid = lax.axis_index("s")

  _zero_f32_ref(zbuf_v, _ZCH)

  @pl.when(sid < NS - 1)
  def _():
    pltpu.sync_copy(zbuf_v.at[pl.ds(0, 624)], deg_sh.at[pl.ds(sid * 624, 624)])

  @pl.when(sid == NS - 1)
  def _():
    pltpu.sync_copy(zbuf_v, deg_sh.at[pl.ds((NS - 1) * 624, _ZCH)])

  plsc.subcore_barrier()

  half = N_CHUNKS // NC
  start = cid * half + jnp.where(sid == 0, 0, 40 + (sid - 1) * 39)
  stop = start + jnp.where(sid == 0, 40, 39)

  @pl.loop(0, 5)
  def _(t):
    j0 = start + t * _DB
    for q in range(_DB):
      @pl.when(j0 + q < stop)
      def _():
        base = (j0 + q) * CHUNK
        pltpu.async_copy(edge_hbm.at[1, pl.ds(base, CHUNK)], cb.at[q], isem)
        pltpu.async_copy(ew_hbm.at[pl.ds(base, CHUNK)], wb.at[q], isem)
    for q in range(_DB):
      @pl.when(j0 + q < stop)
      def _():
        base = (j0 + q) * CHUNK
        pltpu.make_async_copy(edge_hbm.at[1, pl.ds(base, CHUNK)], cb.at[q],
                              isem).wait()
        pltpu.make_async_copy(ew_hbm.at[pl.ds(base, CHUNK)], wb.at[q],
                              isem).wait()
    for q in range(_DB):
      @pl.when(j0 + q < stop)
      def _():
        pltpu.async_copy(wb.at[q], deg_sh.at[cb.at[q]], ssem, add=True)
    for q in range(_DB):
      @pl.when(j0 + q < stop)
      def _():
        pltpu.make_async_copy(wb.at[q], deg_sh.at[cb.at[q]], ssem).wait()

  plsc.subcore_barrier()

  @pl.when(sid == 0)
  def _():
    pltpu.sync_copy(deg_sh, deg2_hbm.at[cid])


_deg_call = pl.kernel(
    _deg_body,
    out_type=jax.ShapeDtypeStruct((NC, N), jnp.float32),
    mesh=_MESH,
    scratch_types=[
        pltpu.VMEM((_DB, CHUNK), jnp.int32),
        pltpu.VMEM((_DB, CHUNK), jnp.float32),
        pltpu.VMEM((_ZCH,), jnp.float32),
        pltpu.VMEM_SHARED((N,), jnp.float32),
        pltpu.SemaphoreType.DMA,
        pltpu.SemaphoreType.DMA,
    ],
)


def _msg_body(h0_hbm, h1_hbm, edge_hbm, ew_hbm, agg0_hbm, agg1_hbm,
              eb0, eb1, eb2, wb0, wb1, wb2, buf0, buf1, buf2, cs0, cs1, cs2,
              agg_sh, isem0, isem1, isem2, gsem0, gsem1, gsem2,
              ssem0, ssem1, ssem2):
  cid = lax.axis_index("c")
  sid = lax.axis_index("s")

  @pl.loop(0, CHUNK)
  def _(r):
    for m in range(HALF // LANES):
      buf0[r, pl.ds(m * LANES, LANES)] = jnp.zeros((LANES,), jnp.float32)

  for q in range(4):
    pltpu.sync_copy(buf0, agg_sh.at[pl.ds(sid * ROWS_PER_TILE + q * CHUNK, CHUNK)])
  pltpu.sync_copy(buf0.at[pl.ds(0, 112)],
                  agg_sh.at[pl.ds(sid * ROWS_PER_TILE + 4 * CHUNK, 112)])

  @pl.when(sid == NS - 1)
  def _():
    pltpu.sync_copy(buf0.at[pl.ds(0, TAIL_ROWS)],
                    agg_sh.at[pl.ds(NS * ROWS_PER_TILE, TAIL_ROWS)])

  plsc.subcore_barrier()

  nchunks = jnp.where(sid < N_CHUNKS % NS, N_CHUNKS // NS + 1, N_CHUNKS // NS)

  ebs, wbs = (eb0, eb1, eb2), (wb0, wb1, wb2)
  bufs, css = (buf0, buf1, buf2), (cs0, cs1, cs2)
  isems, gsems, ssems = (isem0, isem1, isem2), (gsem0, gsem1, gsem2), (
      ssem0, ssem1, ssem2)

  def _base(i):
    return (sid + i * NS) * CHUNK

  def _start_idx(i, b):
    base = _base(i)
    pltpu.async_copy(edge_hbm.at[:, pl.ds(base, CHUNK)], ebs[b], isems[b])
    pltpu.async_copy(ew_hbm.at[pl.ds(base, CHUNK)], wbs[b], isems[b])

  def _wait_idx(i, b):
    base = _base(i)
    pltpu.make_async_copy(edge_hbm.at[:, pl.ds(base, CHUNK)], ebs[b],
                          isems[b]).wait()
    pltpu.make_async_copy(ew_hbm.at[pl.ds(base, CHUNK)], wbs[b],
                          isems[b]).wait()

  HC = CHUNK // 2

  def _start_gather(h_hbm, b):
    pltpu.async_copy(h_hbm.at[ebs[b].at[0, pl.ds(0, HC)]],
                     bufs[b].at[pl.ds(0, HC)], gsems[b])
    pltpu.async_copy(h_hbm.at[ebs[b].at[0, pl.ds(HC, HC)]],
                     bufs[b].at[pl.ds(HC, HC)], gsems[b])

  def _wait_gather(h_hbm, b):
    pltpu.make_async_copy(h_hbm.at[ebs[b].at[0, pl.ds(0, HC)]],
                          bufs[b].at[pl.ds(0, HC)], gsems[b]).wait()
    pltpu.make_async_copy(h_hbm.at[ebs[b].at[0, pl.ds(HC, HC)]],
                          bufs[b].at[pl.ds(HC, HC)], gsems[b]).wait()

  def do_edges(h_hbm, agg_hbm):
    _start_idx(0, 0)
    _start_idx(1, 1)
    _wait_idx(0, 0)
    _start_gather(h_hbm, 0)

    @pl.loop(0, (N_CHUNKS // NS + 3) // 3)
    def _(it):
      for b in range(3):
        i = it * 3 + b
        nx = (b + 1) % 3
        pv = (b + 2) % 3

        @pl.when(i < nchunks)
        def _():
          _wait_gather(h_hbm, b)

          @pl.when(i >= 2)
          def _():
            pltpu.make_async_copy(bufs[nx], agg_sh.at[css[nx]],
                                  ssems[nx]).wait()

          @pl.when(i + 1 < nchunks)
          def _():
            _wait_idx(i + 1, nx)
            _start_gather(h_hbm, nx)

          @pl.loop(0, CHUNK // LANES)
          def _(g):
            w16 = wbs[b][pl.ds(g * LANES, LANES)]
            css[b][pl.ds(g * LANES, LANES)] = ebs[b][1, pl.ds(g * LANES, LANES)]
            for j in range(LANES):
              k = g * LANES + j
              w = w16[j]
              for m in range(HALF // LANES):
                sl = pl.ds(m * LANES, LANES)
                bufs[b][k, sl] = bufs[b][k, sl] * w

          @pl.when(i + 2 < nchunks)
          def _():
            _start_idx(i + 2, pv)

          pltpu.async_copy(bufs[b], agg_sh.at[css[b]], ssems[b], add=True)

    for s in range(3):
      @pl.when(((nchunks - 1) % 3 == s) | ((nchunks - 2) % 3 == s))
      def _():
        pltpu.make_async_copy(bufs[s], agg_sh.at[css[s]], ssems[s]).wait()

  @pl.when(cid == 0)
  def _():
    do_edges(h0_hbm, agg0_hbm)

  @pl.when(cid == 1)
  def _():
    do_edges(h1_hbm, agg1_hbm)

  plsc.subcore_barrier()

  rows = pl.ds(sid * ROWS_PER_TILE, ROWS_PER_TILE)
  tail = pl.ds(NS * ROWS_PER_TILE, TAIL_ROWS)

  @pl.when(cid == 0)
  def _():
    pltpu.sync_copy(agg_sh.at[rows], agg0_hbm.at[rows])

    @pl.when(sid == NS - 1)
    def _():
      pltpu.sync_copy(agg_sh.at[tail], agg0_hbm.at[tail])

  @pl.when(cid == 1)
  def _():
    pltpu.sync_copy(agg_sh.at[rows], agg1_hbm.at[rows])

    @pl.when(sid == NS - 1)
    def _():
      pltpu.sync_copy(agg_sh.at[tail], agg1_hbm.at[tail])


_msg_call = pl.kernel(
    _msg_body,
    out_type=(jax.ShapeDtypeStruct((N, HALF), jnp.float32),
              jax.ShapeDtypeStruct((N, HALF), jnp.float32)),
    mesh=_MESH,
    scratch_types=(
        [pltpu.VMEM((2, CHUNK), jnp.int32)] * 3
        + [pltpu.VMEM((CHUNK,), jnp.float32)] * 3
        + [pltpu.VMEM((CHUNK, HALF), jnp.float32)] * 3
        + [pltpu.VMEM((CHUNK,), jnp.int32)] * 3
        + [pltpu.VMEM_SHARED((N, HALF), jnp.float32)]
        + [pltpu.SemaphoreType.DMA] * 9
    ),
)


ROW_BLK = 1000
_GRID = N // ROW_BLK


def _ln_body(x_ref, deg2_ref, lnw_ref, lnb_ref, h0_ref, h1_ref, dis_ref):
  x = x_ref[...]
  degb = deg2_ref[0]
  deg = degb[0, :] + degb[1, :] + 1.0
  dis = lax.rsqrt(deg)
  mu = jnp.mean(x, axis=1, keepdims=True)
  xc = x - mu
  var = jnp.mean(xc * xc, axis=1, keepdims=True)
  h = xc * lax.rsqrt(var + 1e-5) * lnw_ref[...][None, :] + lnb_ref[...][None, :]
  h = jnp.maximum(h, 0.0) * dis[:, None]
  h0_ref[...] = h[:, :HALF]
  h1_ref[...] = h[:, HALF:]
  dis_ref[...] = dis[:, None]


_ln_call = pl.pallas_call(
    _ln_body,
    grid=(_GRID,),
    in_specs=[
        pl.BlockSpec((ROW_BLK, C), lambda i: (i, 0)),
        pl.BlockSpec((1, NC, ROW_BLK), lambda i: (i, 0, 0)),
        pl.BlockSpec((C,), lambda i: (0,)),
        pl.BlockSpec((C,), lambda i: (0,)),
    ],
    out_specs=(
        pl.BlockSpec((ROW_BLK, HALF), lambda i: (i, 0)),
        pl.BlockSpec((ROW_BLK, HALF), lambda i: (i, 0)),
        pl.BlockSpec((ROW_BLK, 1), lambda i: (i, 0)),
    ),
    out_shape=(
        jax.ShapeDtypeStruct((N, HALF), jnp.float32),
        jax.ShapeDtypeStruct((N, HALF), jnp.float32),
        jax.ShapeDtypeStruct((N, 1), jnp.float32),
    ),
)


def _fin_body(agg0_ref, agg1_ref, h0_ref, h1_ref, dis_ref, x0_ref, w1_ref,
              out_ref):
  dis = dis_ref[...]
  t = jnp.concatenate(
      [agg0_ref[...] + h0_ref[...], agg1_ref[...] + h1_ref[...]], axis=1) * dis
  u = (1.0 - ALPHA) * t + ALPHA * x0_ref[...]
  out_ref[...] = (1.0 - BETA) * u + BETA * jnp.dot(
      u, w1_ref[...], preferred_element_type=jnp.float32)


_fin_call = pl.pallas_call(
    _fin_body,
    grid=(_GRID,),
    in_specs=[
        pl.BlockSpec((ROW_BLK, HALF), lambda i: (i, 0)),
        pl.BlockSpec((ROW_BLK, HALF), lambda i: (i, 0)),
        pl.BlockSpec((ROW_BLK, HALF), lambda i: (i, 0)),
        pl.BlockSpec((ROW_BLK, HALF), lambda i: (i, 0)),
        pl.BlockSpec((ROW_BLK, 1), lambda i: (i, 0)),
        pl.BlockSpec((ROW_BLK, C), lambda i: (i, 0)),
        pl.BlockSpec((C, C), lambda i: (0, 0)),
    ],
    out_specs=pl.BlockSpec((ROW_BLK, C), lambda i: (i, 0)),
    out_shape=jax.ShapeDtypeStruct((N, C), jnp.float32),
)


@jax.jit
def kernel(x, x0, edge_index, edge_weight, ln_weight, ln_bias, weight1):
  deg2 = _deg_call(edge_index, edge_weight)
  deg2t = deg2.reshape(NC, _GRID, ROW_BLK).transpose(1, 0, 2)
  h0, h1, dis = _ln_call(x, deg2t, ln_weight, ln_bias)
  agg0, agg1 = _msg_call(h0, h1, edge_index, edge_weight)
  return _fin_call(agg0, agg1, h0, h1, dis, x0, weight1)


# speedup vs baseline: 1.0045x; 1.0045x over previous
import functools
import math

import jax
import jax.numpy as jnp
from jax import lax
from jax.experimental import pallas as pl
from jax.experimental.pallas import tpu as pltpu
from jax.experimental.pallas import tpu_sc as plsc

N = 10000
E = 160000
C = 256
HALF = 128
ALPHA = 0.5
BETA = math.log(2.0)

NC = 2
NS = 16
LANES = 16
CHUNK = 128
N_CHUNKS = E // CHUNK
ROWS_PER_TILE = 624
TAIL_ROWS = N - NS * ROWS_PER_TILE
ZROWS = 208

_MESH = plsc.VectorSubcoreMesh(core_axis_name="c", subcore_axis_name="s")


def _zero_f32_ref(ref, nwords):
  zeros = jnp.zeros((LANES,), jnp.float32)

  @pl.loop(0, nwords // LANES)
  def _(i):
    ref[pl.ds(i * LANES, LANES)] = zeros


_ZCH = 640
_DB = 8


def _deg_body(edge_hbm, ew_hbm, deg2_hbm, cb, wb, zbuf_v, deg_sh, isem, ssem):
  cid = lax.axis_index("c")
  sid = lax.axis_index("s")

  _zero_f32_ref(zbuf_v, _ZCH)

  @pl.when(sid < NS - 1)
  def _():
    pltpu.sync_copy(zbuf_v.at[pl.ds(0, 624)], deg_sh.at[pl.ds(sid * 624, 624)])

  @pl.when(sid == NS - 1)
  def _():
    pltpu.sync_copy(zbuf_v, deg_sh.at[pl.ds((NS - 1) * 624, _ZCH)])

  plsc.subcore_barrier()

  half = N_CHUNKS // NC
  start = cid * half + jnp.where(sid == 0, 0, 40 + (sid - 1) * 39)
  stop = start + jnp.where(sid == 0, 40, 39)

  @pl.loop(0, 5)
  def _(t):
    j0 = start + t * _DB
    for q in range(_DB):
      @pl.when(j0 + q < stop)
      def _():
        base = (j0 + q) * CHUNK
        pltpu.async_copy(edge_hbm.at[1, pl.ds(base, CHUNK)], cb.at[q], isem)
        pltpu.async_copy(ew_hbm.at[pl.ds(base, CHUNK)], wb.at[q], isem)
    for q in range(_DB):
      @pl.when(j0 + q < stop)
      def _():
        base = (j0 + q) * CHUNK
        pltpu.make_async_copy(edge_hbm.at[1, pl.ds(base, CHUNK)], cb.at[q],
                              isem).wait()
        pltpu.make_async_copy(ew_hbm.at[pl.ds(base, CHUNK)], wb.at[q],
                              isem).wait()
    for q in range(_DB):
      @pl.when(j0 + q < stop)
      def _():
        pltpu.async_copy(wb.at[q], deg_sh.at[cb.at[q]], ssem, add=True)
    for q in range(_DB):
      @pl.when(j0 + q < stop)
      def _():
        pltpu.make_async_copy(wb.at[q], deg_sh.at[cb.at[q]], ssem).wait()

  plsc.subcore_barrier()

  @pl.when(sid == 0)
  def _():
    pltpu.sync_copy(deg_sh, deg2_hbm.at[cid])


_deg_call = pl.kernel(
    _deg_body,
    out_type=jax.ShapeDtypeStruct((NC, N), jnp.float32),
    mesh=_MESH,
    scratch_types=[
        pltpu.VMEM((_DB, CHUNK), jnp.int32),
        pltpu.VMEM((_DB, CHUNK), jnp.float32),
        pltpu.VMEM((_ZCH,), jnp.float32),
        pltpu.VMEM_SHARED((N,), jnp.float32),
        pltpu.SemaphoreType.DMA,
        pltpu.SemaphoreType.DMA,
    ],
)


def _msg_body(h0_hbm, h1_hbm, edge_hbm, ew_hbm, agg0_hbm, agg1_hbm,
              eb0, eb1, eb2, wb0, wb1, wb2, buf0, buf1, buf2, cs0, cs1, cs2,
              agg_sh, isem0, isem1, isem2, gsem0, gsem1, gsem2,
              ssem0, ssem1, ssem2):
  cid = lax.axis_index("c")
  sid = lax.axis_index("s")

  @pl.loop(0, CHUNK)
  def _(r):
    for m in range(HALF // LANES):
      buf0[r, pl.ds(m * LANES, LANES)] = jnp.zeros((LANES,), jnp.float32)

  for q in range(4):
    pltpu.sync_copy(buf0, agg_sh.at[pl.ds(sid * ROWS_PER_TILE + q * CHUNK, CHUNK)])
  pltpu.sync_copy(buf0.at[pl.ds(0, 112)],
                  agg_sh.at[pl.ds(sid * ROWS_PER_TILE + 4 * CHUNK, 112)])

  @pl.when(sid == NS - 1)
  def _():
    pltpu.sync_copy(buf0.at[pl.ds(0, TAIL_ROWS)],
                    agg_sh.at[pl.ds(NS * ROWS_PER_TILE, TAIL_ROWS)])

  plsc.subcore_barrier()

  nchunks = jnp.where(sid < N_CHUNKS % NS, N_CHUNKS // NS + 1, N_CHUNKS // NS)

  ebs, wbs = (eb0, eb1, eb2), (wb0, wb1, wb2)
  bufs, css = (buf0, buf1, buf2), (cs0, cs1, cs2)
  isems, gsems, ssems = (isem0, isem1, isem2), (gsem0, gsem1, gsem2), (
      ssem0, ssem1, ssem2)

  def _base(i):
    return (sid + i * NS) * CHUNK

  def _start_idx(i, b):
    base = _base(i)
    pltpu.async_copy(edge_hbm.at[:, pl.ds(base, CHUNK)], ebs[b], isems[b])
    pltpu.async_copy(ew_hbm.at[pl.ds(base, CHUNK)], wbs[b], isems[b])

  def _wait_idx(i, b):
    base = _base(i)
    pltpu.make_async_copy(edge_hbm.at[:, pl.ds(base, CHUNK)], ebs[b],
                          isems[b]).wait()
    pltpu.make_async_copy(ew_hbm.at[pl.ds(base, CHUNK)], wbs[b],
                          isems[b]).wait()

  def do_edges(h_hbm, agg_hbm):
    _start_idx(0, 0)
    _start_idx(1, 1)
    _wait_idx(0, 0)
    pltpu.async_copy(h_hbm.at[ebs[0].at[0]], bufs[0], gsems[0])

    @pl.loop(0, (N_CHUNKS // NS + 3) // 3)
    def _(it):
      for b in range(3):
        i = it * 3 + b
        nx = (b + 1) % 3
        pv = (b + 2) % 3

        @pl.when(i < nchunks)
        def _():
          pltpu.make_async_copy(h_hbm.at[ebs[b].at[0]], bufs[b],
                                gsems[b]).wait()

          @pl.when(i >= 2)
          def _():
            pltpu.make_async_copy(bufs[nx], agg_sh.at[css[nx]],
                                  ssems[nx]).wait()

          @pl.when(i + 1 < nchunks)
          def _():
            _wait_idx(i + 1, nx)
            pltpu.async_copy(h_hbm.at[ebs[nx].at[0]], bufs[nx], gsems[nx])

          @pl.loop(0, CHUNK // LANES)
          def _(g):
            w16 = wbs[b][pl.ds(g * LANES, LANES)]
            css[b][pl.ds(g * LANES, LANES)] = ebs[b][1, pl.ds(g * LANES, LANES)]
            for j in range(LANES):
              k = g * LANES + j
              w = w16[j]
              for m in range(HALF // LANES):
                sl = pl.ds(m * LANES, LANES)
                bufs[b][k, sl] = bufs[b][k, sl] * w

          @pl.when(i + 2 < nchunks)
          def _():
            _start_idx(i + 2, pv)

          pltpu.async_copy(bufs[b], agg_sh.at[css[b]], ssems[b], add=True)

    for s in range(3):
      @pl.when(((nchunks - 1) % 3 == s) | ((nchunks - 2) % 3 == s))
      def _():
        pltpu.make_async_copy(bufs[s], agg_sh.at[css[s]], ssems[s]).wait()

  @pl.when(cid == 0)
  def _():
    do_edges(h0_hbm, agg0_hbm)

  @pl.when(cid == 1)
  def _():
    do_edges(h1_hbm, agg1_hbm)

  plsc.subcore_barrier()

  rows = pl.ds(sid * ROWS_PER_TILE, ROWS_PER_TILE)
  tail = pl.ds(NS * ROWS_PER_TILE, TAIL_ROWS)

  @pl.when(cid == 0)
  def _():
    pltpu.sync_copy(agg_sh.at[rows], agg0_hbm.at[rows])

    @pl.when(sid == NS - 1)
    def _():
      pltpu.sync_copy(agg_sh.at[tail], agg0_hbm.at[tail])

  @pl.when(cid == 1)
  def _():
    pltpu.sync_copy(agg_sh.at[rows], agg1_hbm.at[rows])

    @pl.when(sid == NS - 1)
    def _():
      pltpu.sync_copy(agg_sh.at[tail], agg1_hbm.at[tail])


_msg_call = pl.kernel(
    _msg_body,
    out_type=(jax.ShapeDtypeStruct((N, HALF), jnp.float32),
              jax.ShapeDtypeStruct((N, HALF), jnp.float32)),
    mesh=_MESH,
    scratch_types=(
        [pltpu.VMEM((2, CHUNK), jnp.int32)] * 3
        + [pltpu.VMEM((CHUNK,), jnp.float32)] * 3
        + [pltpu.VMEM((CHUNK, HALF), jnp.float32)] * 3
        + [pltpu.VMEM((CHUNK,), jnp.int32)] * 3
        + [pltpu.VMEM_SHARED((N, HALF), jnp.float32)]
        + [pltpu.SemaphoreType.DMA] * 9
    ),
)


ROW_BLK = 1000
_GRID = N // ROW_BLK


def _ln_body(x_ref, deg2_ref, lnw_ref, lnb_ref, h0_ref, h1_ref, dis_ref):
  x = x_ref[...]
  degb = deg2_ref[0]
  deg = degb[0, :] + degb[1, :] + 1.0
  dis = lax.rsqrt(deg)
  mu = jnp.mean(x, axis=1, keepdims=True)
  xc = x - mu
  var = jnp.mean(xc * xc, axis=1, keepdims=True)
  h = xc * lax.rsqrt(var + 1e-5) * lnw_ref[...][None, :] + lnb_ref[...][None, :]
  h = jnp.maximum(h, 0.0) * dis[:, None]
  h0_ref[...] = h[:, :HALF]
  h1_ref[...] = h[:, HALF:]
  dis_ref[...] = dis[:, None]


_ln_call = pl.pallas_call(
    _ln_body,
    grid=(_GRID,),
    in_specs=[
        pl.BlockSpec((ROW_BLK, C), lambda i: (i, 0)),
        pl.BlockSpec((1, NC, ROW_BLK), lambda i: (i, 0, 0)),
        pl.BlockSpec((C,), lambda i: (0,)),
        pl.BlockSpec((C,), lambda i: (0,)),
    ],
    out_specs=(
        pl.BlockSpec((ROW_BLK, HALF), lambda i: (i, 0)),
        pl.BlockSpec((ROW_BLK, HALF), lambda i: (i, 0)),
        pl.BlockSpec((ROW_BLK, 1), lambda i: (i, 0)),
    ),
    out_shape=(
        jax.ShapeDtypeStruct((N, HALF), jnp.float32),
        jax.ShapeDtypeStruct((N, HALF), jnp.float32),
        jax.ShapeDtypeStruct((N, 1), jnp.float32),
    ),
)


def _fin_body(agg0_ref, agg1_ref, h0_ref, h1_ref, dis_ref, x0_ref, w1_ref,
              out_ref):
  dis = dis_ref[...]
  t = jnp.concatenate(
      [agg0_ref[...] + h0_ref[...], agg1_ref[...] + h1_ref[...]], axis=1) * dis
  u = (1.0 - ALPHA) * t + ALPHA * x0_ref[...]
  out_ref[...] = (1.0 - BETA) * u + BETA * jnp.dot(
      u, w1_ref[...], preferred_element_type=jnp.float32)


_fin_call = pl.pallas_call(
    _fin_body,
    grid=(_GRID,),
    in_specs=[
        pl.BlockSpec((ROW_BLK, HALF), lambda i: (i, 0)),
        pl.BlockSpec((ROW_BLK, HALF), lambda i: (i, 0)),
        pl.BlockSpec((ROW_BLK, HALF), lambda i: (i, 0)),
        pl.BlockSpec((ROW_BLK, HALF), lambda i: (i, 0)),
        pl.BlockSpec((ROW_BLK, 1), lambda i: (i, 0)),
        pl.BlockSpec((ROW_BLK, C), lambda i: (i, 0)),
        pl.BlockSpec((C, C), lambda i: (0, 0)),
    ],
    out_specs=pl.BlockSpec((ROW_BLK, C), lambda i: (i, 0)),
    out_shape=jax.ShapeDtypeStruct((N, C), jnp.float32),
)


@jax.jit
def kernel(x, x0, edge_index, edge_weight, ln_weight, ln_bias, weight1):
  deg2 = _deg_call(edge_index, edge_weight)
  deg2t = deg2.reshape(NC, _GRID, ROW_BLK).transpose(1, 0, 2)
  h0, h1, dis = _ln_call(x, deg2t, ln_weight, ln_bias)
  agg0, agg1 = _msg_call(h0, h1, edge_index, edge_weight)
  return _fin_call(agg0, agg1, h0, h1, dis, x0, weight1)
